# baseline (speedup 1.0000x reference)
import functools

import jax
import jax.numpy as jnp
from jax import lax
from jax.experimental import pallas as pl
from jax.experimental.pallas import tpu as pltpu
from jax.experimental.pallas import tpu_sc as plsc

NC = 2
NS = 16
L = 16
NW = NC * NS
B = 16384
D = 64
BPW = B // NW
CHUNK = 128
NCHUNK = BPW // CHUNK
NBLK = BPW // L


def _score_body(users_hbm, items_hbm, ub_hbm, ib_hbm, ue_hbm, ie_hbm,
                out_hbm, uidx, iidx, u_rows, i_rows, ub, ib, out_v, tsc,
                sem):
    wid = lax.axis_index("s") * NC + lax.axis_index("c")
    base = wid * BPW

    pltpu.sync_copy(users_hbm.at[pl.ds(wid * NCHUNK, NCHUNK)], uidx)
    pltpu.sync_copy(items_hbm.at[pl.ds(wid * NCHUNK, NCHUNK)], iidx)

    copies = []
    for k in range(NCHUNK):
        copies.append(pltpu.async_copy(
            ue_hbm.at[uidx.at[k]], u_rows.at[pl.ds(k * CHUNK, CHUNK)], sem))
        copies.append(pltpu.async_copy(
            ie_hbm.at[iidx.at[k]], i_rows.at[pl.ds(k * CHUNK, CHUNK)], sem))
        copies.append(pltpu.async_copy(
            ub_hbm.at[uidx.at[k]], ub.at[pl.ds(k * CHUNK, CHUNK)], sem))
        copies.append(pltpu.async_copy(
            ib_hbm.at[iidx.at[k]], ib.at[pl.ds(k * CHUNK, CHUNK)], sem))
    for c in copies:
        c.wait()

    rows17 = lax.iota(jnp.int32, L) * (L + 1)

    def block(b, carry):
        e0 = pl.multiple_of(b * L, L)
        acc = ub[pl.ds(e0, L)] + ib[pl.ds(e0, L)]
        for e in range(L):
            er = e0 + e
            s = u_rows[er, pl.ds(0, L)] * i_rows[er, pl.ds(0, L)]
            for q in range(1, D // L):
                s = s + u_rows[er, pl.ds(q * L, L)] * i_rows[er, pl.ds(q * L, L)]
            tsc[pl.ds(e * (L + 1), L)] = s
        for j in range(L):
            col = plsc.load_gather(tsc, [rows17 + j])
            acc = acc + col
        out_v[pl.ds(e0, L)] = acc
        return carry

    lax.fori_loop(0, NBLK, block, 0)
    pltpu.sync_copy(out_v, out_hbm.at[pl.ds(base, BPW)])


_score_kernel = functools.partial(
    pl.kernel,
    out_type=jax.ShapeDtypeStruct((B,), jnp.float32),
    mesh=plsc.VectorSubcoreMesh(core_axis_name="c", subcore_axis_name="s"),
    compiler_params=pltpu.CompilerParams(
        needs_layout_passes=False, use_tc_tiling_on_sc=False),
    scratch_types=[
        pltpu.VMEM((NCHUNK, CHUNK), jnp.int32),
        pltpu.VMEM((NCHUNK, CHUNK), jnp.int32),
        pltpu.VMEM((BPW, D), jnp.float32),
        pltpu.VMEM((BPW, D), jnp.float32),
        pltpu.VMEM((BPW,), jnp.float32),
        pltpu.VMEM((BPW,), jnp.float32),
        pltpu.VMEM((BPW,), jnp.float32),
        pltpu.VMEM((L * (L + 1),), jnp.float32),
        pltpu.SemaphoreType.DMA,
    ],
)(_score_body)


def kernel(users, items, u_bias_w, i_bias_w, u_embed_w, i_embed_w):
    users2d = users.astype(jnp.int32).reshape(NW * NCHUNK, CHUNK)
    items2d = items.astype(jnp.int32).reshape(NW * NCHUNK, CHUNK)
    return _score_kernel(
        users2d, items2d,
        u_bias_w.reshape(-1), i_bias_w.reshape(-1),
        u_embed_w, i_embed_w)
